# Initial kernel scaffold; baseline (speedup 1.0000x reference)
import jax
import jax.numpy as jnp
from jax.experimental import pallas as pl


def kernel(inputs):
    raise NotImplementedError("write your pallas kernel here")


# speedup vs baseline: 47.1924x; 47.1924x over previous
import jax
import jax.numpy as jnp
from jax.experimental import pallas as pl

_N = 1000000
_V = 32
_B = 32768


def _vote_block(x_ref, o_ref):
    s = jnp.sum(x_ref[...], axis=0)
    o_ref[...] = (s > _V // 2).astype(jnp.int32)


def kernel(inputs):
    n_blocks = (_N + _B - 1) // _B
    out = pl.pallas_call(
        _vote_block,
        grid=(n_blocks,),
        in_specs=[pl.BlockSpec((_V, _B), lambda i: (0, i))],
        out_specs=pl.BlockSpec((_B,), lambda i: (i,)),
        out_shape=jax.ShapeDtypeStruct((_N,), jnp.int32),
    )(inputs)
    return out
